# baseline (speedup 1.0000x reference)
import jax
import jax.numpy as jnp
from jax import lax
from jax.experimental import pallas as pl
from jax.experimental.pallas import tpu as pltpu
from jax.experimental.pallas import tpu_sc as plsc

_B = 16384
_D = 64
_K = 20
_S = _K + 1
_NC = 2
_NS = 16
_NW = _NC * _NS
_CHUNK = _B // _NW
_NSTEP = 16
_BS = _CHUNK // _NSTEP
_LANES = 16
_DV = _D // _LANES
_NG = _BS // _LANES


def _sc_scores_body(cidx_hbm, kidx_hbm, cpar_hbm, kpar_hbm,
                    in_hbm, out_hbm, scores_hbm,
                    cidx_v, kidx_v, cpar_v, kpar_v,
                    crow_v, orow_v, scores_v, sem):
    wid = lax.axis_index("s") * _NC + lax.axis_index("c")
    lane = lax.iota(jnp.int32, _LANES)
    cols = [lane + 16 * k for k in range(_DV)]

    def step(t, carry):
        pltpu.sync_copy(cidx_hbm.at[wid, t], cidx_v)
        pltpu.sync_copy(kidx_hbm.at[wid, t], kidx_v)
        pltpu.sync_copy(cpar_hbm.at[wid, t], cpar_v)
        pltpu.sync_copy(kpar_hbm.at[wid, t], kpar_v)
        descs = [pltpu.async_copy(in_hbm.at[cidx_v], crow_v, sem)]
        for j in range(_S):
            descs.append(
                pltpu.async_copy(out_hbm.at[kidx_v.at[j]], orow_v.at[j], sem))
        for d in descs:
            d.wait()

        def group(g, carry):
            gsl = pl.ds(g * _LANES, _LANES)
            cpar_g = cpar_v[gsl]
            kpar_g = [kpar_v[j, gsl] for j in range(_S)]

            def dots(b, res):
                bb = jnp.full((_LANES,), g * _LANES + b, jnp.int32)
                cp = jnp.take_along_axis(cpar_g, bb, axis=0,
                                         mode="promise_in_bounds")
                c = [plsc.load_gather(crow_v, [bb, cp + col]) for col in cols]
                cn = [-ck for ck in c]
                new = []
                for j in range(_S):
                    cc = c if j == 0 else cn
                    kp = jnp.take_along_axis(kpar_g[j], bb, axis=0,
                                             mode="promise_in_bounds")
                    oj = orow_v.at[j]
                    acc = cc[0] * plsc.load_gather(oj, [bb, kp + cols[0]])
                    for k in range(1, _DV):
                        acc = acc + cc[k] * plsc.load_gather(
                            oj, [bb, kp + cols[k]])
                    s = jnp.sum(acc)
                    new.append(jnp.where(lane == b, s, res[j]))
                return tuple(new)

            res = lax.fori_loop(
                0, _LANES, dots,
                tuple(jnp.zeros((_LANES,), jnp.float32) for _ in range(_S)))
            base = t * _BS + g * _LANES
            for j in range(_S):
                scores_v[j, pl.ds(base, _LANES)] = res[j]
            return carry

        return lax.fori_loop(0, _NG, group, carry)

    lax.fori_loop(0, _NSTEP, step, 0)

    pltpu.sync_copy(scores_v,
                    scores_hbm.at[:, pl.ds(wid * _CHUNK, _CHUNK)])


def _sc_scores():
    return pl.kernel(
        _sc_scores_body,
        out_type=jax.ShapeDtypeStruct((_S, _B), jnp.float32),
        mesh=plsc.VectorSubcoreMesh(
            core_axis_name="c", subcore_axis_name="s",
            num_cores=_NC, num_subcores=_NS),
        compiler_params=pltpu.CompilerParams(
            needs_layout_passes=False, use_tc_tiling_on_sc=True),
        scratch_types=[
            pltpu.VMEM((_BS,), jnp.int32),
            pltpu.VMEM((_S, _BS), jnp.int32),
            pltpu.VMEM((_BS,), jnp.int32),
            pltpu.VMEM((_S, _BS), jnp.int32),
            pltpu.VMEM((_BS, 2 * _D), jnp.float32),
            pltpu.VMEM((_S, _BS, 2 * _D), jnp.float32),
            pltpu.VMEM((_S, _CHUNK), jnp.float32),
            pltpu.SemaphoreType.DMA,
        ],
    )


def _loss_body(s_ref, o_ref):
    i = pl.program_id(0)
    x = s_ref[...]
    ls = jnp.minimum(x, 0.0) - jnp.log1p(jnp.exp(-jnp.abs(x)))
    part = -jnp.sum(ls) / _B

    @pl.when(i == 0)
    def _():
        o_ref[0, 0] = 0.0

    o_ref[0, 0] += part


_COLS_PER_BLOCK = 2048


def _tc_loss(scores):
    out = pl.pallas_call(
        _loss_body,
        grid=(_B // _COLS_PER_BLOCK,),
        in_specs=[pl.BlockSpec((_S, _COLS_PER_BLOCK), lambda i: (0, i))],
        out_specs=pl.BlockSpec(memory_space=pltpu.SMEM),
        out_shape=jax.ShapeDtypeStruct((1, 1), jnp.float32),
    )(scores)
    return out[0, 0]


def kernel(center_idx, context_idx, neg_idx, in_emb, out_emb):
    in2 = in_emb.reshape(in_emb.shape[0] // 2, 2 * _D)
    out2 = out_emb.reshape(out_emb.shape[0] // 2, 2 * _D)
    comb = jnp.concatenate([context_idx[None, :], neg_idx.T], axis=0)
    cidx = (center_idx >> 1).reshape(_NW, _NSTEP, _BS)
    cpar = ((center_idx & 1) * _D).reshape(_NW, _NSTEP, _BS)
    kidx = (comb >> 1).reshape(_S, _NW, _NSTEP, _BS).transpose(1, 2, 0, 3)
    kpar = ((comb & 1) * _D).reshape(_S, _NW, _NSTEP, _BS).transpose(1, 2, 0, 3)
    scores = _sc_scores()(cidx, kidx, cpar, kpar, in2, out2)
    return _tc_loss(scores)


# speedup vs baseline: 1.0870x; 1.0870x over previous
import jax
import jax.numpy as jnp
from jax import lax
from jax.experimental import pallas as pl
from jax.experimental.pallas import tpu as pltpu
from jax.experimental.pallas import tpu_sc as plsc

_B = 16384
_D = 64
_K = 20
_S = _K + 1
_NC = 2
_NS = 16
_NW = _NC * _NS
_CHUNK = _B // _NW
_NSTEP = 16
_BS = _CHUNK // _NSTEP
_LANES = 16
_DV = _D // _LANES
_NG = _BS // _LANES


def _sc_scores_body(cidx_hbm, kidx_hbm, cpar_hbm, kpar_hbm,
                    in_hbm, out_hbm, scores_hbm,
                    cidx_v, kidx_v, cpar_v, kpar_v,
                    crow_v, orow_v, scores_v, sem):
    wid = lax.axis_index("s") * _NC + lax.axis_index("c")
    lane = lax.iota(jnp.int32, _LANES)

    def step(t, carry):
        pltpu.sync_copy(cidx_hbm.at[wid, t], cidx_v)
        pltpu.sync_copy(kidx_hbm.at[wid, t], kidx_v)
        pltpu.sync_copy(cpar_hbm.at[wid, t], cpar_v)
        pltpu.sync_copy(kpar_hbm.at[wid, t], kpar_v)
        descs = [pltpu.async_copy(in_hbm.at[cidx_v], crow_v, sem)]
        for j in range(_S):
            descs.append(
                pltpu.async_copy(out_hbm.at[kidx_v.at[j]], orow_v.at[j], sem))
        for d in descs:
            d.wait()

        def group(g, carry):
            cpvec = cpar_v[pl.ds(g * _LANES, _LANES)]
            kpv = [kpar_v[pl.ds(j * _BS + g * _LANES, _LANES)]
                   for j in range(_S)]

            def dots(b, res):
                bb = g * _LANES + b
                sel = lane == b
                cp = jnp.sum(jnp.where(sel, cpvec, 0))
                c = [crow_v[bb, pl.ds(cp + 16 * k, 16)] for k in range(_DV)]
                cn = [-ck for ck in c]
                new = []
                for j in range(_S):
                    cc = c if j == 0 else cn
                    kp = jnp.sum(jnp.where(sel, kpv[j], 0))
                    acc = cc[0] * orow_v[j, bb, pl.ds(kp, 16)]
                    for k in range(1, _DV):
                        acc = acc + cc[k] * orow_v[j, bb, pl.ds(kp + 16 * k, 16)]
                    s = jnp.sum(acc)
                    new.append(jnp.where(sel, s, res[j]))
                return tuple(new)

            res = lax.fori_loop(
                0, _LANES, dots,
                tuple(jnp.zeros((_LANES,), jnp.float32) for _ in range(_S)))
            base = t * _BS + g * _LANES
            for j in range(_S):
                scores_v[j, pl.ds(base, _LANES)] = res[j]
            return carry

        return lax.fori_loop(0, _NG, group, carry)

    lax.fori_loop(0, _NSTEP, step, 0)

    pltpu.sync_copy(scores_v,
                    scores_hbm.at[:, pl.ds(wid * _CHUNK, _CHUNK)])


def _sc_scores():
    return pl.kernel(
        _sc_scores_body,
        out_type=jax.ShapeDtypeStruct((_S, _B), jnp.float32),
        mesh=plsc.VectorSubcoreMesh(
            core_axis_name="c", subcore_axis_name="s",
            num_cores=_NC, num_subcores=_NS),
        compiler_params=pltpu.CompilerParams(
            needs_layout_passes=False, use_tc_tiling_on_sc=True),
        scratch_types=[
            pltpu.VMEM((_BS,), jnp.int32),
            pltpu.VMEM((_S, _BS), jnp.int32),
            pltpu.VMEM((_BS,), jnp.int32),
            pltpu.VMEM((_S * _BS,), jnp.int32),
            pltpu.VMEM((_BS, 2 * _D), jnp.float32),
            pltpu.VMEM((_S, _BS, 2 * _D), jnp.float32),
            pltpu.VMEM((_S, _CHUNK), jnp.float32),
            pltpu.SemaphoreType.DMA,
        ],
    )


def _loss_body(s_ref, o_ref):
    i = pl.program_id(0)
    x = s_ref[...]
    ls = jnp.minimum(x, 0.0) - jnp.log1p(jnp.exp(-jnp.abs(x)))
    part = -jnp.sum(ls) / _B

    @pl.when(i == 0)
    def _():
        o_ref[0, 0] = 0.0

    o_ref[0, 0] += part


_COLS_PER_BLOCK = 2048


def _tc_loss(scores):
    out = pl.pallas_call(
        _loss_body,
        grid=(_B // _COLS_PER_BLOCK,),
        in_specs=[pl.BlockSpec((_S, _COLS_PER_BLOCK), lambda i: (0, i))],
        out_specs=pl.BlockSpec(memory_space=pltpu.SMEM),
        out_shape=jax.ShapeDtypeStruct((1, 1), jnp.float32),
    )(scores)
    return out[0, 0]


def kernel(center_idx, context_idx, neg_idx, in_emb, out_emb):
    in2 = in_emb.reshape(in_emb.shape[0] // 2, 2 * _D)
    out2 = out_emb.reshape(out_emb.shape[0] // 2, 2 * _D)
    comb = jnp.concatenate([context_idx[None, :], neg_idx.T], axis=0)
    cidx = (center_idx >> 1).reshape(_NW, _NSTEP, _BS)
    cpar = ((center_idx & 1) * _D).reshape(_NW, _NSTEP, _BS)
    kidx = (comb >> 1).reshape(_S, _NW, _NSTEP, _BS).transpose(1, 2, 0, 3)
    kpar = ((comb & 1) * _D).reshape(_S, _NW, _NSTEP, _BS)
    kpar = kpar.transpose(1, 2, 0, 3).reshape(_NW, _NSTEP, _S * _BS)
    scores = _sc_scores()(cidx, kidx, cpar, kpar, in2, out2)
    return _tc_loss(scores)


# speedup vs baseline: 1.2453x; 1.1457x over previous
import jax
import jax.numpy as jnp
from jax import lax
from jax.experimental import pallas as pl
from jax.experimental.pallas import tpu as pltpu
from jax.experimental.pallas import tpu_sc as plsc

_B = 16384
_D = 64
_K = 20
_S = _K + 1
_NC = 2
_NS = 16
_NW = _NC * _NS
_CHUNK = _B // _NW
_NSTEP = 16
_BS = _CHUNK // _NSTEP
_LANES = 16
_DV = _D // _LANES
_NG = _BS // _LANES


def _sc_scores_body(cidx_hbm, kidx_hbm, cpar_hbm, kpar_hbm,
                    in_hbm, out_hbm, scores_hbm,
                    cidx_v, kidx_v, cpar_v, kpar_v,
                    crow_v, orow_v, scores_v, sem):
    wid = lax.axis_index("s") * _NC + lax.axis_index("c")
    lane = lax.iota(jnp.int32, _LANES)

    def step(t, carry):
        pltpu.sync_copy(cidx_hbm.at[wid, t], cidx_v)
        pltpu.sync_copy(kidx_hbm.at[wid, t], kidx_v)
        pltpu.sync_copy(cpar_hbm.at[wid, t], cpar_v)
        pltpu.sync_copy(kpar_hbm.at[wid, t], kpar_v)
        descs = [pltpu.async_copy(in_hbm.at[cidx_v], crow_v, sem)]
        for j in range(_S):
            descs.append(
                pltpu.async_copy(out_hbm.at[kidx_v.at[j]], orow_v.at[j], sem))
        for d in descs:
            d.wait()

        def group(g, carry):
            cpvec = cpar_v[pl.ds(g * _LANES, _LANES)]
            kpv = [kpar_v[pl.ds(j * _BS + g * _LANES, _LANES)]
                   for j in range(_S)]

            def dots(b, res):
                bb = g * _LANES + b
                sel = lane == b
                cp = jnp.sum(jnp.where(sel, cpvec, 0))
                c = [crow_v[bb, pl.ds(cp + 16 * k, 16)] for k in range(_DV)]
                cn = [-ck for ck in c]
                new = []
                for j in range(_S):
                    cc = c if j == 0 else cn
                    kp = jnp.sum(jnp.where(sel, kpv[j], 0))
                    acc = cc[0] * orow_v[j, bb, pl.ds(kp, 16)]
                    for k in range(1, _DV):
                        acc = acc + cc[k] * orow_v[j, bb, pl.ds(kp + 16 * k, 16)]
                    s = jnp.sum(acc)
                    new.append(jnp.where(sel, s, res[j]))
                return tuple(new)

            res = lax.fori_loop(
                0, _LANES, dots,
                tuple(jnp.zeros((_LANES,), jnp.float32) for _ in range(_S)))
            base = t * _BS + g * _LANES
            for j in range(_S):
                scores_v[j, pl.ds(base, _LANES)] = res[j]
            return carry

        return lax.fori_loop(0, _NG, group, carry)

    lax.fori_loop(0, _NSTEP, step, 0)

    pltpu.sync_copy(scores_v,
                    scores_hbm.at[:, pl.ds(wid * _CHUNK, _CHUNK)])


def _sc_scores():
    return pl.kernel(
        _sc_scores_body,
        out_type=jax.ShapeDtypeStruct((_S, _B), jnp.float32),
        mesh=plsc.VectorSubcoreMesh(
            core_axis_name="c", subcore_axis_name="s",
            num_cores=_NC, num_subcores=_NS),
        compiler_params=pltpu.CompilerParams(
            needs_layout_passes=False, use_tc_tiling_on_sc=True),
        scratch_types=[
            pltpu.VMEM((_BS,), jnp.int32),
            pltpu.VMEM((_S, _BS), jnp.int32),
            pltpu.VMEM((_BS,), jnp.int32),
            pltpu.VMEM((_S * _BS,), jnp.int32),
            pltpu.VMEM((_BS, 2 * _D), jnp.float32),
            pltpu.VMEM((_S, _BS, 2 * _D), jnp.float32),
            pltpu.VMEM((_S, _CHUNK), jnp.float32),
            pltpu.SemaphoreType.DMA,
        ],
    )


def _loss_body(s_ref, o_ref):
    i = pl.program_id(0)
    x = s_ref[...]
    ls = jnp.minimum(x, 0.0) - jnp.log1p(jnp.exp(-jnp.abs(x)))
    part = -jnp.sum(ls) / _B

    @pl.when(i == 0)
    def _():
        o_ref[0, 0] = 0.0

    o_ref[0, 0] += part


_COLS_PER_BLOCK = 2048
_VB = 1024
_NCONV = 489
_VROWS = _NCONV * _VB


def _conv_body(a_ref, b_ref, o_ref):
    za = jnp.transpose(a_ref[...])
    zb = jnp.transpose(b_ref[...])
    o_ref[...] = jnp.concatenate([za, zb], axis=1)


def _tc_convert(embT):
    return pl.pallas_call(
        _conv_body,
        grid=(_NCONV,),
        in_specs=[
            pl.BlockSpec((_D, _VB), lambda i: (0, 2 * i)),
            pl.BlockSpec((_D, _VB),
                         lambda i: (0, jnp.minimum(2 * i + 1, 2 * _NCONV - 2))),
        ],
        out_specs=pl.BlockSpec((_VB, 2 * _D), lambda i: (i, 0)),
        out_shape=jax.ShapeDtypeStruct((_VROWS, 2 * _D), jnp.float32),
    )(embT, embT)


def _tc_loss(scores):
    out = pl.pallas_call(
        _loss_body,
        grid=(_B // _COLS_PER_BLOCK,),
        in_specs=[pl.BlockSpec((_S, _COLS_PER_BLOCK), lambda i: (0, i))],
        out_specs=pl.BlockSpec(memory_space=pltpu.SMEM),
        out_shape=jax.ShapeDtypeStruct((1, 1), jnp.float32),
    )(scores)
    return out[0, 0]


def kernel(center_idx, context_idx, neg_idx, in_emb, out_emb):
    in2 = _tc_convert(in_emb.T)
    out2 = _tc_convert(out_emb.T)
    comb = jnp.concatenate([context_idx[None, :], neg_idx.T], axis=0)
    crow = ((center_idx >> 11) << 10) + (center_idx & 1023)
    chalf = ((center_idx >> 10) & 1) * _D
    krow = ((comb >> 11) << 10) + (comb & 1023)
    khalf = ((comb >> 10) & 1) * _D
    cidx = crow.reshape(_NW, _NSTEP, _BS)
    cpar = chalf.reshape(_NW, _NSTEP, _BS)
    kidx = krow.reshape(_S, _NW, _NSTEP, _BS).transpose(1, 2, 0, 3)
    kpar = khalf.reshape(_S, _NW, _NSTEP, _BS)
    kpar = kpar.transpose(1, 2, 0, 3).reshape(_NW, _NSTEP, _S * _BS)
    scores = _sc_scores()(cidx, kidx, cpar, kpar, in2, out2)
    return _tc_loss(scores)


# speedup vs baseline: 1.7095x; 1.3728x over previous
import jax
import jax.numpy as jnp
from jax import lax
from jax.experimental import pallas as pl
from jax.experimental.pallas import tpu as pltpu
from jax.experimental.pallas import tpu_sc as plsc

_B = 16384
_D = 64
_K = 20
_S = _K + 1
_NC = 2
_NS = 16
_NW = _NC * _NS
_CHUNK = _B // _NW
_NSTEP = 16
_BS = _CHUNK // _NSTEP
_LANES = 16
_DV = _D // _LANES
_NG = _BS // _LANES


def _sc_scores_body(cidx_hbm, kidx_hbm, cpar_hbm, kpar_hbm,
                    in_hbm, out_hbm, scores_hbm,
                    cidx_v, kidx_v, cpar_v, kpar_v,
                    crow_v, orow_v, scores_v, sem):
    wid = lax.axis_index("s") * _NC + lax.axis_index("c")
    lane = lax.iota(jnp.int32, _LANES)

    def step(t, carry):
        pltpu.sync_copy(cidx_hbm.at[wid, t], cidx_v)
        pltpu.sync_copy(kidx_hbm.at[wid, t], kidx_v)
        pltpu.sync_copy(cpar_hbm.at[wid, t], cpar_v)
        pltpu.sync_copy(kpar_hbm.at[wid, t], kpar_v)
        descs = [pltpu.async_copy(in_hbm.at[cidx_v], crow_v, sem)]
        for j in range(_S):
            descs.append(
                pltpu.async_copy(out_hbm.at[kidx_v.at[j]], orow_v.at[j], sem))
        for d in descs:
            d.wait()

        def group(g, carry):
            cpvec = cpar_v[pl.ds(g * _LANES, _LANES)]
            kpv = [kpar_v[pl.ds(j * _BS + g * _LANES, _LANES)]
                   for j in range(_S)]

            def dots(b, res):
                bb = g * _LANES + b
                sel = lane == b
                cp = jnp.sum(jnp.where(sel, cpvec, 0))
                c = [crow_v[bb, pl.ds(cp + 16 * k, 16)] for k in range(_DV)]
                cn = [-ck for ck in c]
                new = []
                for j in range(_S):
                    cc = c if j == 0 else cn
                    lo = cc[0] * orow_v[j, bb, pl.ds(0, 16)]
                    hi = cc[0] * orow_v[j, bb, pl.ds(_D, 16)]
                    for k in range(1, _DV):
                        lo = lo + cc[k] * orow_v[j, bb, pl.ds(16 * k, 16)]
                        hi = hi + cc[k] * orow_v[j, bb, pl.ds(_D + 16 * k, 16)]
                    s_lo = jnp.sum(lo)
                    s_hi = jnp.sum(hi)
                    m_lo = sel & (kpv[j] == 0)
                    new.append(
                        jnp.where(m_lo, s_lo, jnp.where(sel, s_hi, res[j])))
                return tuple(new)

            res = lax.fori_loop(
                0, _LANES, dots,
                tuple(jnp.zeros((_LANES,), jnp.float32) for _ in range(_S)))
            base = t * _BS + g * _LANES
            for j in range(_S):
                scores_v[j, pl.ds(base, _LANES)] = res[j]
            return carry

        return lax.fori_loop(0, _NG, group, carry)

    lax.fori_loop(0, _NSTEP, step, 0)

    pltpu.sync_copy(scores_v,
                    scores_hbm.at[:, pl.ds(wid * _CHUNK, _CHUNK)])


def _sc_scores():
    return pl.kernel(
        _sc_scores_body,
        out_type=jax.ShapeDtypeStruct((_S, _B), jnp.float32),
        mesh=plsc.VectorSubcoreMesh(
            core_axis_name="c", subcore_axis_name="s",
            num_cores=_NC, num_subcores=_NS),
        compiler_params=pltpu.CompilerParams(
            needs_layout_passes=False, use_tc_tiling_on_sc=True),
        scratch_types=[
            pltpu.VMEM((_BS,), jnp.int32),
            pltpu.VMEM((_S, _BS), jnp.int32),
            pltpu.VMEM((_BS,), jnp.int32),
            pltpu.VMEM((_S * _BS,), jnp.int32),
            pltpu.VMEM((_BS, 2 * _D), jnp.float32),
            pltpu.VMEM((_S, _BS, 2 * _D), jnp.float32),
            pltpu.VMEM((_S, _CHUNK), jnp.float32),
            pltpu.SemaphoreType.DMA,
        ],
    )


def _loss_body(s_ref, o_ref):
    i = pl.program_id(0)
    x = s_ref[...]
    ls = jnp.minimum(x, 0.0) - jnp.log1p(jnp.exp(-jnp.abs(x)))
    part = -jnp.sum(ls) / _B

    @pl.when(i == 0)
    def _():
        o_ref[0, 0] = 0.0

    o_ref[0, 0] += part


_COLS_PER_BLOCK = 2048
_VB = 2048
_NCONV = 245
_VROWS = _NCONV * _VB


def _conv_body(a_ref, b_ref, o_ref):
    za = jnp.transpose(a_ref[...])
    zb = jnp.transpose(b_ref[...])
    o_ref[...] = jnp.concatenate([za, zb], axis=1)


def _tc_convert(embT):
    return pl.pallas_call(
        _conv_body,
        grid=(_NCONV,),
        in_specs=[
            pl.BlockSpec((_D, _VB), lambda i: (0, 2 * i)),
            pl.BlockSpec((_D, _VB),
                         lambda i: (0, jnp.minimum(2 * i + 1, 2 * _NCONV - 2))),
        ],
        out_specs=pl.BlockSpec((_VB, 2 * _D), lambda i: (i, 0)),
        out_shape=jax.ShapeDtypeStruct((_VROWS, 2 * _D), jnp.float32),
    )(embT, embT)


def _tc_loss(scores):
    out = pl.pallas_call(
        _loss_body,
        grid=(_B // _COLS_PER_BLOCK,),
        in_specs=[pl.BlockSpec((_S, _COLS_PER_BLOCK), lambda i: (0, i))],
        out_specs=pl.BlockSpec(memory_space=pltpu.SMEM),
        out_shape=jax.ShapeDtypeStruct((1, 1), jnp.float32),
    )(scores)
    return out[0, 0]


def kernel(center_idx, context_idx, neg_idx, in_emb, out_emb):
    in2 = _tc_convert(in_emb.T)
    out2 = _tc_convert(out_emb.T)
    comb = jnp.concatenate([context_idx[None, :], neg_idx.T], axis=0)
    crow = ((center_idx >> 12) << 11) + (center_idx & 2047)
    chalf = ((center_idx >> 11) & 1) * _D
    krow = ((comb >> 12) << 11) + (comb & 2047)
    khalf = ((comb >> 11) & 1) * _D
    cidx = crow.reshape(_NW, _NSTEP, _BS)
    cpar = chalf.reshape(_NW, _NSTEP, _BS)
    kidx = krow.reshape(_S, _NW, _NSTEP, _BS).transpose(1, 2, 0, 3)
    kpar = khalf.reshape(_S, _NW, _NSTEP, _BS)
    kpar = kpar.transpose(1, 2, 0, 3).reshape(_NW, _NSTEP, _S * _BS)
    scores = _sc_scores()(cidx, kidx, cpar, kpar, in2, out2)
    return _tc_loss(scores)


# speedup vs baseline: 2.0365x; 1.1912x over previous
import jax
import jax.numpy as jnp
from jax import lax
from jax.experimental import pallas as pl
from jax.experimental.pallas import tpu as pltpu
from jax.experimental.pallas import tpu_sc as plsc

_B = 16384
_D = 64
_K = 20
_S = _K + 1
_NC = 2
_NS = 16
_NW = _NC * _NS
_CHUNK = _B // _NW
_NSTEP = 16
_BS = _CHUNK // _NSTEP
_LANES = 16
_DV = _D // _LANES
_NG = _BS // _LANES


def _sc_scores_body(cidx_hbm, kidx_hbm, cpar_hbm, kpar_hbm,
                    in_hbm, out_hbm, scores_hbm,
                    cidx_v, kidx_v, cpar_v, kpar_v,
                    crow_v, orow_v, scores_v, sem):
    wid = lax.axis_index("s") * _NC + lax.axis_index("c")
    lane = lax.iota(jnp.int32, _LANES)

    def step(t, carry):
        pltpu.sync_copy(cidx_hbm.at[wid, t], cidx_v)
        pltpu.sync_copy(kidx_hbm.at[wid, t], kidx_v)
        pltpu.sync_copy(cpar_hbm.at[wid, t], cpar_v)
        pltpu.sync_copy(kpar_hbm.at[wid, t], kpar_v)
        descs = [pltpu.async_copy(in_hbm.at[cidx_v], crow_v, sem)]
        for j in range(_S):
            descs.append(
                pltpu.async_copy(out_hbm.at[kidx_v.at[j]], orow_v.at[j], sem))
        for d in descs:
            d.wait()

        def group(g, carry):
            cpvec = cpar_v[pl.ds(g * _LANES, _LANES)]
            kpv = [kpar_v[pl.ds(j * _BS + g * _LANES, _LANES)]
                   for j in range(_S)]

            def dots(b, res):
                bb = g * _LANES + b
                sel = lane == b
                cp = jnp.sum(jnp.where(sel, cpvec, 0))
                c = [crow_v[bb, pl.ds(cp + 16 * k, 16)] for k in range(_DV)]
                cn = [-ck for ck in c]
                new = []
                for j in range(_S):
                    cc = c if j == 0 else cn
                    lo = cc[0] * orow_v[j, bb, pl.ds(0, 16)]
                    hi = cc[0] * orow_v[j, bb, pl.ds(_D, 16)]
                    for k in range(1, _DV):
                        lo = lo + cc[k] * orow_v[j, bb, pl.ds(16 * k, 16)]
                        hi = hi + cc[k] * orow_v[j, bb, pl.ds(_D + 16 * k, 16)]
                    s_lo = jnp.sum(lo)
                    s_hi = jnp.sum(hi)
                    m_lo = sel & (kpv[j] == 0)
                    new.append(
                        jnp.where(m_lo, s_lo, jnp.where(sel, s_hi, res[j])))
                return tuple(new)

            res = lax.fori_loop(
                0, _LANES, dots,
                tuple(jnp.zeros((_LANES,), jnp.float32) for _ in range(_S)))
            base = t * _BS + g * _LANES
            for j in range(_S):
                scores_v[j, pl.ds(base, _LANES)] = res[j]
            return carry

        return lax.fori_loop(0, _NG, group, carry)

    lax.fori_loop(0, _NSTEP, step, 0)

    pltpu.sync_copy(scores_v,
                    scores_hbm.at[:, pl.ds(wid * _CHUNK, _CHUNK)])


def _sc_scores():
    return pl.kernel(
        _sc_scores_body,
        out_type=jax.ShapeDtypeStruct((_S, _B), jnp.float32),
        mesh=plsc.VectorSubcoreMesh(
            core_axis_name="c", subcore_axis_name="s",
            num_cores=_NC, num_subcores=_NS),
        compiler_params=pltpu.CompilerParams(
            needs_layout_passes=False, use_tc_tiling_on_sc=True),
        scratch_types=[
            pltpu.VMEM((_BS,), jnp.int32),
            pltpu.VMEM((_S, _BS), jnp.int32),
            pltpu.VMEM((_BS,), jnp.int32),
            pltpu.VMEM((_S * _BS,), jnp.int32),
            pltpu.VMEM((_BS, 2 * _D), jnp.float32),
            pltpu.VMEM((_S, _BS, 2 * _D), jnp.float32),
            pltpu.VMEM((_S, _CHUNK), jnp.float32),
            pltpu.SemaphoreType.DMA,
        ],
    )


def _loss_body(s_ref, o_ref):
    i = pl.program_id(0)
    x = s_ref[...]
    ls = jnp.minimum(x, 0.0) - jnp.log1p(jnp.exp(-jnp.abs(x)))
    part = -jnp.sum(ls) / _B

    @pl.when(i == 0)
    def _():
        o_ref[0, 0] = 0.0

    o_ref[0, 0] += part


_COLS_PER_BLOCK = 2048
_VB = 4096
_NCONV = 123
_VROWS = _NCONV * _VB


def _conv_body(a_ref, b_ref, o_ref):
    za = jnp.transpose(a_ref[...])
    zb = jnp.transpose(b_ref[...])
    o_ref[...] = jnp.concatenate([za, zb], axis=1)


def _tc_convert(embT):
    return pl.pallas_call(
        _conv_body,
        grid=(_NCONV,),
        in_specs=[
            pl.BlockSpec((_D, _VB), lambda i: (0, 2 * i)),
            pl.BlockSpec((_D, _VB),
                         lambda i: (0, jnp.minimum(2 * i + 1, 2 * _NCONV - 2))),
        ],
        out_specs=pl.BlockSpec((_VB, 2 * _D), lambda i: (i, 0)),
        out_shape=jax.ShapeDtypeStruct((_VROWS, 2 * _D), jnp.float32),
    )(embT, embT)


def _tc_loss(scores):
    out = pl.pallas_call(
        _loss_body,
        grid=(_B // _COLS_PER_BLOCK,),
        in_specs=[pl.BlockSpec((_S, _COLS_PER_BLOCK), lambda i: (0, i))],
        out_specs=pl.BlockSpec(memory_space=pltpu.SMEM),
        out_shape=jax.ShapeDtypeStruct((1, 1), jnp.float32),
    )(scores)
    return out[0, 0]


def kernel(center_idx, context_idx, neg_idx, in_emb, out_emb):
    in2 = _tc_convert(in_emb.T)
    out2 = _tc_convert(out_emb.T)
    comb = jnp.concatenate([context_idx[None, :], neg_idx.T], axis=0)
    crow = ((center_idx >> 13) << 12) + (center_idx & 4095)
    chalf = ((center_idx >> 12) & 1) * _D
    krow = ((comb >> 13) << 12) + (comb & 4095)
    khalf = ((comb >> 12) & 1) * _D
    cidx = crow.reshape(_NW, _NSTEP, _BS)
    cpar = chalf.reshape(_NW, _NSTEP, _BS)
    kidx = krow.reshape(_S, _NW, _NSTEP, _BS).transpose(1, 2, 0, 3)
    kpar = khalf.reshape(_S, _NW, _NSTEP, _BS)
    kpar = kpar.transpose(1, 2, 0, 3).reshape(_NW, _NSTEP, _S * _BS)
    scores = _sc_scores()(cidx, kidx, cpar, kpar, in2, out2)
    return _tc_loss(scores)


# speedup vs baseline: 2.2269x; 1.0935x over previous
import jax
import jax.numpy as jnp
from jax import lax
from jax.experimental import pallas as pl
from jax.experimental.pallas import tpu as pltpu
from jax.experimental.pallas import tpu_sc as plsc

_B = 16384
_D = 64
_K = 20
_S = _K + 1
_NC = 2
_NS = 16
_NW = _NC * _NS
_CHUNK = _B // _NW
_NSTEP = 16
_BS = _CHUNK // _NSTEP
_LANES = 16
_DV = _D // _LANES
_NG = _BS // _LANES


def _sc_scores_body(cidx_hbm, kidx_hbm, cpar_hbm, kpar_hbm,
                    in_hbm, out_hbm, scores_hbm,
                    cidx_v, kidx_v, cpar_v, kpar_v,
                    crow_v, orow_v, scores_v, sem):
    wid = lax.axis_index("s") * _NC + lax.axis_index("c")
    lane = lax.iota(jnp.int32, _LANES)

    def step(t, carry):
        pltpu.sync_copy(cidx_hbm.at[wid, t], cidx_v)
        pltpu.sync_copy(kidx_hbm.at[wid, t], kidx_v)
        pltpu.sync_copy(cpar_hbm.at[wid, t], cpar_v)
        pltpu.sync_copy(kpar_hbm.at[wid, t], kpar_v)
        descs = [pltpu.async_copy(in_hbm.at[cidx_v], crow_v, sem)]
        for j in range(_S):
            descs.append(
                pltpu.async_copy(out_hbm.at[kidx_v.at[j]], orow_v.at[j], sem))
        for d in descs:
            d.wait()

        def group(g, carry):
            cpvec = cpar_v[pl.ds(g * _LANES, _LANES)]
            kpv = [kpar_v[pl.ds(j * _BS + g * _LANES, _LANES)]
                   for j in range(_S)]

            def dots(b, res):
                bb = g * _LANES + b
                sel = lane == b
                cp = jnp.sum(jnp.where(sel, cpvec, 0))
                c = [crow_v[bb, pl.ds(cp + 16 * k, 16)] for k in range(_DV)]
                cn = [-ck for ck in c]
                new = []
                for j in range(_S):
                    cc = c if j == 0 else cn
                    lo = cc[0] * orow_v[j, bb, pl.ds(0, 16)]
                    hi = cc[0] * orow_v[j, bb, pl.ds(_D, 16)]
                    for k in range(1, _DV):
                        lo = lo + cc[k] * orow_v[j, bb, pl.ds(16 * k, 16)]
                        hi = hi + cc[k] * orow_v[j, bb, pl.ds(_D + 16 * k, 16)]
                    s_lo = jnp.sum(lo)
                    s_hi = jnp.sum(hi)
                    m_lo = sel & (kpv[j] == 0)
                    new.append(
                        jnp.where(m_lo, s_lo, jnp.where(sel, s_hi, res[j])))
                return tuple(new)

            res = lax.fori_loop(
                0, _LANES, dots,
                tuple(jnp.zeros((_LANES,), jnp.float32) for _ in range(_S)))
            base = t * _BS + g * _LANES
            for j in range(_S):
                scores_v[j, pl.ds(base, _LANES)] = res[j]
            return carry

        return lax.fori_loop(0, _NG, group, carry)

    lax.fori_loop(0, _NSTEP, step, 0)

    pltpu.sync_copy(scores_v,
                    scores_hbm.at[:, pl.ds(wid * _CHUNK, _CHUNK)])


def _sc_scores():
    return pl.kernel(
        _sc_scores_body,
        out_type=jax.ShapeDtypeStruct((_S, _B), jnp.float32),
        mesh=plsc.VectorSubcoreMesh(
            core_axis_name="c", subcore_axis_name="s",
            num_cores=_NC, num_subcores=_NS),
        compiler_params=pltpu.CompilerParams(
            needs_layout_passes=False, use_tc_tiling_on_sc=True),
        scratch_types=[
            pltpu.VMEM((_BS,), jnp.int32),
            pltpu.VMEM((_S, _BS), jnp.int32),
            pltpu.VMEM((_BS,), jnp.int32),
            pltpu.VMEM((_S * _BS,), jnp.int32),
            pltpu.VMEM((_BS, 2 * _D), jnp.float32),
            pltpu.VMEM((_S, _BS, 2 * _D), jnp.float32),
            pltpu.VMEM((_S, _CHUNK), jnp.float32),
            pltpu.SemaphoreType.DMA,
        ],
    )


def _loss_body(s_ref, o_ref):
    i = pl.program_id(0)
    x = s_ref[...]
    ls = jnp.minimum(x, 0.0) - jnp.log1p(jnp.exp(-jnp.abs(x)))
    part = -jnp.sum(ls) / _B

    @pl.when(i == 0)
    def _():
        o_ref[0, 0] = 0.0

    o_ref[0, 0] += part


_COLS_PER_BLOCK = 2048
_VB = 8192
_NCONV = 62
_VROWS = _NCONV * _VB


def _conv_body(a_ref, b_ref, o_ref):
    za = jnp.transpose(a_ref[...])
    zb = jnp.transpose(b_ref[...])
    o_ref[...] = jnp.concatenate([za, zb], axis=1)


def _tc_convert(embT):
    return pl.pallas_call(
        _conv_body,
        grid=(_NCONV,),
        in_specs=[
            pl.BlockSpec((_D, _VB), lambda i: (0, 2 * i)),
            pl.BlockSpec((_D, _VB),
                         lambda i: (0, jnp.minimum(2 * i + 1, 2 * _NCONV - 2))),
        ],
        out_specs=pl.BlockSpec((_VB, 2 * _D), lambda i: (i, 0)),
        out_shape=jax.ShapeDtypeStruct((_VROWS, 2 * _D), jnp.float32),
    )(embT, embT)


def _tc_loss(scores):
    out = pl.pallas_call(
        _loss_body,
        grid=(_B // _COLS_PER_BLOCK,),
        in_specs=[pl.BlockSpec((_S, _COLS_PER_BLOCK), lambda i: (0, i))],
        out_specs=pl.BlockSpec(memory_space=pltpu.SMEM),
        out_shape=jax.ShapeDtypeStruct((1, 1), jnp.float32),
    )(scores)
    return out[0, 0]


def kernel(center_idx, context_idx, neg_idx, in_emb, out_emb):
    in2 = _tc_convert(in_emb.T)
    out2 = _tc_convert(out_emb.T)
    comb = jnp.concatenate([context_idx[None, :], neg_idx.T], axis=0)
    crow = ((center_idx >> 14) << 13) + (center_idx & 8191)
    chalf = ((center_idx >> 13) & 1) * _D
    krow = ((comb >> 14) << 13) + (comb & 8191)
    khalf = ((comb >> 13) & 1) * _D
    cidx = crow.reshape(_NW, _NSTEP, _BS)
    cpar = chalf.reshape(_NW, _NSTEP, _BS)
    kidx = krow.reshape(_S, _NW, _NSTEP, _BS).transpose(1, 2, 0, 3)
    kpar = khalf.reshape(_S, _NW, _NSTEP, _BS)
    kpar = kpar.transpose(1, 2, 0, 3).reshape(_NW, _NSTEP, _S * _BS)
    scores = _sc_scores()(cidx, kidx, cpar, kpar, in2, out2)
    return _tc_loss(scores)


# speedup vs baseline: 2.3310x; 1.0468x over previous
import jax
import jax.numpy as jnp
from jax import lax
from jax.experimental import pallas as pl
from jax.experimental.pallas import tpu as pltpu
from jax.experimental.pallas import tpu_sc as plsc

_B = 16384
_D = 64
_K = 20
_S = _K + 1
_NC = 2
_NS = 16
_NW = _NC * _NS
_CHUNK = _B // _NW
_NSTEP = 16
_BS = _CHUNK // _NSTEP
_LANES = 16
_DV = _D // _LANES
_NG = _BS // _LANES


def _sc_scores_body(cidx_hbm, kidx_hbm, cpar_hbm, kpar_hbm,
                    in_hbm, out_hbm, scores_hbm,
                    cidx_v, kidx_v, cpar_v, kpar_v,
                    crow_v, orow_v, scores_v, sem):
    wid = lax.axis_index("s") * _NC + lax.axis_index("c")
    lane = lax.iota(jnp.int32, _LANES)

    def step(t, carry):
        pltpu.sync_copy(cidx_hbm.at[wid, t], cidx_v)
        pltpu.sync_copy(kidx_hbm.at[wid, t], kidx_v)
        pltpu.sync_copy(cpar_hbm.at[wid, t], cpar_v)
        pltpu.sync_copy(kpar_hbm.at[wid, t], kpar_v)
        descs = [pltpu.async_copy(in_hbm.at[cidx_v], crow_v, sem)]
        for j in range(_S):
            descs.append(
                pltpu.async_copy(out_hbm.at[kidx_v.at[j]], orow_v.at[j], sem))
        for d in descs:
            d.wait()

        def group(g, carry):
            cpvec = cpar_v[pl.ds(g * _LANES, _LANES)]
            kpv = [kpar_v[pl.ds(j * _BS + g * _LANES, _LANES)]
                   for j in range(_S)]

            def dots(b, res):
                bb = g * _LANES + b
                sel = lane == b
                cp = jnp.sum(jnp.where(sel, cpvec, 0))
                c = [crow_v[bb, pl.ds(cp + 16 * k, 16)] for k in range(_DV)]
                cn = [-ck for ck in c]
                new = []
                for j in range(_S):
                    cc = c if j == 0 else cn
                    lo = cc[0] * orow_v[j, bb, pl.ds(0, 16)]
                    hi = cc[0] * orow_v[j, bb, pl.ds(_D, 16)]
                    for k in range(1, _DV):
                        lo = lo + cc[k] * orow_v[j, bb, pl.ds(16 * k, 16)]
                        hi = hi + cc[k] * orow_v[j, bb, pl.ds(_D + 16 * k, 16)]
                    s_lo = jnp.sum(lo)
                    s_hi = jnp.sum(hi)
                    m_lo = sel & (kpv[j] == 0)
                    new.append(
                        jnp.where(m_lo, s_lo, jnp.where(sel, s_hi, res[j])))
                return tuple(new)

            res = lax.fori_loop(
                0, _LANES, dots,
                tuple(jnp.zeros((_LANES,), jnp.float32) for _ in range(_S)))
            base = t * _BS + g * _LANES
            for j in range(_S):
                scores_v[j, pl.ds(base, _LANES)] = res[j]
            return carry

        return lax.fori_loop(0, _NG, group, carry)

    lax.fori_loop(0, _NSTEP, step, 0)

    pltpu.sync_copy(scores_v,
                    scores_hbm.at[:, pl.ds(wid * _CHUNK, _CHUNK)])


def _sc_scores():
    return pl.kernel(
        _sc_scores_body,
        out_type=jax.ShapeDtypeStruct((_S, _B), jnp.float32),
        mesh=plsc.VectorSubcoreMesh(
            core_axis_name="c", subcore_axis_name="s",
            num_cores=_NC, num_subcores=_NS),
        compiler_params=pltpu.CompilerParams(
            needs_layout_passes=False, use_tc_tiling_on_sc=True),
        scratch_types=[
            pltpu.VMEM((_BS,), jnp.int32),
            pltpu.VMEM((_S, _BS), jnp.int32),
            pltpu.VMEM((_BS,), jnp.int32),
            pltpu.VMEM((_S * _BS,), jnp.int32),
            pltpu.VMEM((_BS, 2 * _D), jnp.float32),
            pltpu.VMEM((_S, _BS, 2 * _D), jnp.float32),
            pltpu.VMEM((_S, _CHUNK), jnp.float32),
            pltpu.SemaphoreType.DMA,
        ],
    )


def _loss_body(s_ref, o_ref):
    i = pl.program_id(0)
    x = s_ref[...]
    ls = jnp.minimum(x, 0.0) - jnp.log1p(jnp.exp(-jnp.abs(x)))
    part = -jnp.sum(ls) / _B

    @pl.when(i == 0)
    def _():
        o_ref[0, 0] = 0.0

    o_ref[0, 0] += part


_COLS_PER_BLOCK = 2048
_VB = 16384
_NCONV = 31
_VROWS = _NCONV * _VB


def _conv_body(a_ref, b_ref, o_ref):
    za = jnp.transpose(a_ref[...])
    zb = jnp.transpose(b_ref[...])
    o_ref[...] = jnp.concatenate([za, zb], axis=1)


def _tc_convert(embT):
    return pl.pallas_call(
        _conv_body,
        grid=(_NCONV,),
        in_specs=[
            pl.BlockSpec((_D, _VB), lambda i: (0, 2 * i)),
            pl.BlockSpec((_D, _VB),
                         lambda i: (0, jnp.minimum(2 * i + 1, 2 * _NCONV - 1))),
        ],
        out_specs=pl.BlockSpec((_VB, 2 * _D), lambda i: (i, 0)),
        out_shape=jax.ShapeDtypeStruct((_VROWS, 2 * _D), jnp.float32),
    )(embT, embT)


def _tc_loss(scores):
    out = pl.pallas_call(
        _loss_body,
        grid=(_B // _COLS_PER_BLOCK,),
        in_specs=[pl.BlockSpec((_S, _COLS_PER_BLOCK), lambda i: (0, i))],
        out_specs=pl.BlockSpec(memory_space=pltpu.SMEM),
        out_shape=jax.ShapeDtypeStruct((1, 1), jnp.float32),
    )(scores)
    return out[0, 0]


def kernel(center_idx, context_idx, neg_idx, in_emb, out_emb):
    in2 = _tc_convert(in_emb.T)
    out2 = _tc_convert(out_emb.T)
    comb = jnp.concatenate([context_idx[None, :], neg_idx.T], axis=0)
    crow = ((center_idx >> 15) << 14) + (center_idx & 16383)
    chalf = ((center_idx >> 14) & 1) * _D
    krow = ((comb >> 15) << 14) + (comb & 16383)
    khalf = ((comb >> 14) & 1) * _D
    cidx = crow.reshape(_NW, _NSTEP, _BS)
    cpar = chalf.reshape(_NW, _NSTEP, _BS)
    kidx = krow.reshape(_S, _NW, _NSTEP, _BS).transpose(1, 2, 0, 3)
    kpar = khalf.reshape(_S, _NW, _NSTEP, _BS)
    kpar = kpar.transpose(1, 2, 0, 3).reshape(_NW, _NSTEP, _S * _BS)
    scores = _sc_scores()(cidx, kidx, cpar, kpar, in2, out2)
    return _tc_loss(scores)
